# baseline (speedup 1.0000x reference)
import functools

import jax
import jax.numpy as jnp
from jax import lax
from jax.experimental import pallas as pl
from jax.experimental.pallas import tpu as pltpu
from jax.experimental.pallas import tpu_sc as plsc

_NUM_CORES = 2
_NUM_SUBCORES = 16
_NUM_WORKERS = _NUM_CORES * _NUM_SUBCORES

_DIM = 300
_CHUNK = 256
_TAIL = _DIM - _CHUNK
_VOCAB_TILE = 4096


def _sc_gather256(table, words):
    batch = words.shape[0]
    b_per_w = batch // _NUM_WORKERS
    mesh = plsc.VectorSubcoreMesh(core_axis_name="c", subcore_axis_name="s")

    @functools.partial(
        pl.kernel,
        mesh=mesh,
        out_type=jax.ShapeDtypeStruct((batch, _CHUNK), jnp.float32),
        scratch_types=[
            pltpu.VMEM((b_per_w,), jnp.int32),
            pltpu.VMEM((b_per_w, _CHUNK), jnp.float32),
            pltpu.SemaphoreType.DMA,
        ],
    )
    def gather_kernel(table_hbm, idx_hbm, out_hbm, idx_v, rows_v, sem):
        wid = lax.axis_index("s") * _NUM_CORES + lax.axis_index("c")
        base = wid * b_per_w
        pltpu.sync_copy(idx_hbm.at[pl.ds(base, b_per_w)], idx_v)
        c0 = pltpu.async_copy(
            table_hbm.at[idx_v, pl.ds(0, 128)], rows_v.at[:, pl.ds(0, 128)], sem
        )
        c1 = pltpu.async_copy(
            table_hbm.at[idx_v, pl.ds(128, 128)], rows_v.at[:, pl.ds(128, 128)], sem
        )
        c0.wait()
        c1.wait()
        pltpu.sync_copy(rows_v, out_hbm.at[pl.ds(base, b_per_w)])

    return gather_kernel(table, words)


def _norm_matmul_body(
    words_ref, e_ref, w_ref, b_ref, table_ref, o_ref, en_ref, tail_ref, sem
):
    j = pl.program_id(0)
    i = pl.program_id(1)
    batch = e_ref.shape[0]
    half = batch // 2

    @pl.when(jnp.logical_and(j == 0, i == 0))
    def _():
        def issue(i, c):
            pltpu.make_async_copy(
                table_ref.at[pl.ds(words_ref[i], 1), pl.ds(_CHUNK, _TAIL)],
                tail_ref.at[pl.ds(i, 1), pl.ds(_CHUNK, _TAIL)],
                sem,
            ).start()
            return c

        lax.fori_loop(0, batch, issue, 0)
        tail_ref[:, :_CHUNK] = e_ref[...]
        pltpu.make_async_copy(
            table_ref.at[pl.ds(0, batch), pl.ds(_CHUNK, _TAIL)],
            tail_ref.at[:, pl.ds(_CHUNK, _TAIL)],
            sem,
        ).wait()
        e = tail_ref[...]
        ss = jnp.sum(e * e, axis=1, keepdims=True)
        norm = jnp.sqrt(ss)
        scale = jnp.minimum(1.0, 1.0 / jnp.maximum(norm, 1e-7))
        en_ref[...] = (e * scale).astype(jnp.bfloat16)

    w = w_ref[...].astype(jnp.bfloat16)
    en = en_ref[pl.ds(i * half, half), :]
    acc = lax.dot_general(
        en, w, (((1,), (1,)), ((), ())), preferred_element_type=jnp.float32
    )
    o_ref[...] = acc + b_ref[...]


def _tc_norm_matmul(embed256, words, table, W, b):
    batch = embed256.shape[0]
    vocab = W.shape[0]
    nv = pl.cdiv(vocab, _VOCAB_TILE)
    b2 = b.reshape(1, vocab)
    return pl.pallas_call(
        _norm_matmul_body,
        grid=(nv, 2),
        in_specs=[
            pl.BlockSpec(memory_space=pltpu.SMEM),
            pl.BlockSpec((batch, _CHUNK), lambda j, i: (0, 0)),
            pl.BlockSpec((_VOCAB_TILE, _DIM), lambda j, i: (j, 0)),
            pl.BlockSpec((1, _VOCAB_TILE), lambda j, i: (0, j)),
            pl.BlockSpec(memory_space=pl.ANY),
        ],
        out_specs=pl.BlockSpec((batch // 2, _VOCAB_TILE), lambda j, i: (i, j)),
        out_shape=jax.ShapeDtypeStruct((batch, vocab), jnp.float32),
        scratch_shapes=[
            pltpu.VMEM((batch, _DIM), jnp.bfloat16),
            pltpu.VMEM((batch, _DIM), jnp.float32),
            pltpu.SemaphoreType.DMA,
        ],
        compiler_params=pltpu.CompilerParams(
            dimension_semantics=("arbitrary", "arbitrary"),
        ),
    )(words, embed256, W, b2, table)


def kernel(words, table, W, b):
    wi = words.astype(jnp.int32)
    embed256 = _sc_gather256(table, wi)
    return _tc_norm_matmul(embed256, wi, table, W, b)


# speedup vs baseline: 1.0433x; 1.0433x over previous
import functools

import jax
import jax.numpy as jnp
from jax import lax
from jax.experimental import pallas as pl
from jax.experimental.pallas import tpu as pltpu
from jax.experimental.pallas import tpu_sc as plsc

_NUM_CORES = 2
_NUM_SUBCORES = 16
_NUM_WORKERS = _NUM_CORES * _NUM_SUBCORES

_DIM = 300
_CHUNK = 256
_TAIL = _DIM - _CHUNK
_VOCAB_TILE = 4096


def _sc_gather256(table, words):
    batch = words.shape[0]
    b_per_w = batch // _NUM_WORKERS
    mesh = plsc.VectorSubcoreMesh(core_axis_name="c", subcore_axis_name="s")

    @functools.partial(
        pl.kernel,
        mesh=mesh,
        out_type=jax.ShapeDtypeStruct((batch, _CHUNK), jnp.float32),
        scratch_types=[
            pltpu.VMEM((b_per_w,), jnp.int32),
            pltpu.VMEM((b_per_w, _CHUNK), jnp.float32),
            pltpu.SemaphoreType.DMA,
        ],
    )
    def gather_kernel(table_hbm, idx_hbm, out_hbm, idx_v, rows_v, sem):
        wid = lax.axis_index("s") * _NUM_CORES + lax.axis_index("c")
        base = wid * b_per_w
        pltpu.sync_copy(idx_hbm.at[pl.ds(base, b_per_w)], idx_v)
        pltpu.async_copy(
            table_hbm.at[idx_v, pl.ds(0, _CHUNK)], rows_v, sem
        ).wait()
        pltpu.sync_copy(rows_v, out_hbm.at[pl.ds(base, b_per_w)])

    return gather_kernel(table, words)


def _norm_matmul_body(
    words_ref, e_ref, w_ref, b_ref, table_ref, o_ref, en_ref, tail_ref, sem
):
    j = pl.program_id(0)
    batch = e_ref.shape[0]

    @pl.when(j == 0)
    def _():
        def issue(i, c):
            pltpu.make_async_copy(
                table_ref.at[pl.ds(words_ref[i], 1), pl.ds(_CHUNK, _TAIL)],
                tail_ref.at[pl.ds(i, 1), pl.ds(_CHUNK, _TAIL)],
                sem,
            ).start()
            return c

        lax.fori_loop(0, batch, issue, 0)
        tail_ref[:, :_CHUNK] = e_ref[...]
        pltpu.make_async_copy(
            table_ref.at[pl.ds(0, batch), pl.ds(_CHUNK, _TAIL)],
            tail_ref.at[:, pl.ds(_CHUNK, _TAIL)],
            sem,
        ).wait()
        e = tail_ref[...]
        ss = jnp.sum(e * e, axis=1, keepdims=True)
        norm = jnp.sqrt(ss)
        scale = jnp.minimum(1.0, 1.0 / jnp.maximum(norm, 1e-7))
        en_ref[...] = (e * scale).astype(jnp.bfloat16)

    w = w_ref[...].astype(jnp.bfloat16)
    acc = lax.dot_general(
        en_ref[...], w, (((1,), (1,)), ((), ())), preferred_element_type=jnp.float32
    )
    o_ref[...] = acc + b_ref[...]


def _tc_norm_matmul(embed256, words, table, W, b):
    batch = embed256.shape[0]
    vocab = W.shape[0]
    nv = pl.cdiv(vocab, _VOCAB_TILE)
    b2 = b.reshape(1, vocab)
    return pl.pallas_call(
        _norm_matmul_body,
        grid=(nv,),
        in_specs=[
            pl.BlockSpec(memory_space=pltpu.SMEM),
            pl.BlockSpec((batch, _CHUNK), lambda j: (0, 0)),
            pl.BlockSpec((_VOCAB_TILE, _DIM), lambda j: (j, 0)),
            pl.BlockSpec((1, _VOCAB_TILE), lambda j: (0, j)),
            pl.BlockSpec(memory_space=pl.ANY),
        ],
        out_specs=pl.BlockSpec((batch, _VOCAB_TILE), lambda j: (0, j)),
        out_shape=jax.ShapeDtypeStruct((batch, vocab), jnp.float32),
        scratch_shapes=[
            pltpu.VMEM((batch, _DIM), jnp.bfloat16),
            pltpu.VMEM((batch, _DIM), jnp.float32),
            pltpu.SemaphoreType.DMA,
        ],
        compiler_params=pltpu.CompilerParams(
            dimension_semantics=("arbitrary",),
        ),
    )(words, embed256, W, b2, table)


def kernel(words, table, W, b):
    wi = words.astype(jnp.int32)
    embed256 = _sc_gather256(table, wi)
    return _tc_norm_matmul(embed256, wi, table, W, b)
